# baseline (speedup 1.0000x reference)
import functools
import math

import jax
import jax.numpy as jnp
from jax import lax
from jax.experimental import pallas as pl
from jax.experimental.pallas import tpu as pltpu
from jax.experimental.pallas import tpu_sc as plsc

N = 10000
E = 320000
G = 16
NPG = 625
NPP = 640
H = 128
EPS = 1e-5
K_LIST = [313, 157, 79, 40]

GSZ = NPP * NPP
CHUNK = 4 * GSZ
TRASH = 8192
REGION = CHUNK + TRASH
OSLICE = CHUNK // 16
ZBUF = 3200
TSLICE = TRASH // 16

EPT = E // 16
CE = 4096
NCH = 5
CE_TAIL = EPT - (NCH - 1) * CE
CROWS = CE // 128
MAGIC = 26844


def _sc_build_adjacency(edge_index):
  mesh = plsc.VectorSubcoreMesh(core_axis_name="c", subcore_axis_name="s")

  @functools.partial(
      pl.kernel,
      out_type=jax.ShapeDtypeStruct((G * GSZ,), jnp.float32),
      mesh=mesh,
      scratch_types=[
          pltpu.VMEM((2 * CE,), jnp.int32),
          pltpu.VMEM((2 * CE,), jnp.int32),
          pltpu.VMEM((2, CROWS, 128), jnp.int32),
          pltpu.VMEM((128,), jnp.float32),
          pltpu.VMEM((ZBUF,), jnp.float32),
          pltpu.VMEM_SHARED((REGION,), jnp.float32),
          pltpu.SemaphoreType.DMA,
          pltpu.SemaphoreType.DMA,
      ],
  )
  def build(edge_hbm, out_hbm, src_c, dst_c, idx_c,
            ones_v, zeros_v, acc_sh, sem, esem):
    c = lax.axis_index("c")
    s = lax.axis_index("s")
    lane = lax.iota(jnp.int32, 16)

    for i in range(8):
      ones_v[pl.ds(i * 16, 16)] = jnp.ones((16,), jnp.float32)

    def zfill(i, _):
      zeros_v[pl.ds(i * 16, 16)] = jnp.zeros((16,), jnp.float32)
      return ()
    lax.fori_loop(0, ZBUF // 16, zfill, ())

    def zero_own_slices():
      for z in range(32):
        pltpu.sync_copy(zeros_v, acc_sh.at[pl.ds(s * OSLICE + z * ZBUF, ZBUF)])
      pltpu.sync_copy(zeros_v.at[pl.ds(0, TSLICE)],
                      acc_sh.at[pl.ds(CHUNK + s * TSLICE, TSLICE)])

    def fetch(ci, eb):
      csz = CE_TAIL if ci == NCH - 1 else CE
      ebase = s * EPT + ci * CE
      return [
          pltpu.async_copy(edge_hbm.at[pl.ds(ebase, csz)],
                           src_c.at[pl.ds(eb * CE, csz)], esem),
          pltpu.async_copy(edge_hbm.at[pl.ds(E + ebase, csz)],
                           dst_c.at[pl.ds(eb * CE, csz)], esem),
      ]

    zero_own_slices()
    plsc.subcore_barrier()

    for p in range(2):
      chunk = c * 2 + p
      base_flat = chunk * CHUNK

      efetch = {0: fetch(0, 0)}
      sdescs = {}
      for ci in range(NCH):
        eb = ci % 2
        for d in efetch.pop(ci % 2):
          d.wait()
        if ci + 1 < NCH:
          efetch[(ci + 1) % 2] = fetch(ci + 1, (ci + 1) % 2)

        for d in sdescs.pop(eb, []):
          d.wait()

        def idx_body(jj, _):
          for u in range(4):
            j = jj * 4 + u
            e = j * 16
            sv = src_c[pl.ds(eb * CE + e, 16)]
            dv = dst_c[pl.ds(eb * CE + e, 16)]
            g = (dv * MAGIC) >> 24
            ld = dv - g * NPG
            ls = sv - ((sv * MAGIC) >> 24) * NPG
            tiled = ((ls >> 7) * 81920 + ((ld >> 3) << 10)
                     + ((ld & 7) << 7) + (ls & 127))
            flat = g * GSZ + tiled
            loc = flat - base_flat
            eid = ci * CE + e + lane
            inb = (eid < EPT) & (loc >= 0) & (loc < CHUNK)
            tr = CHUNK + ((eid + s * 1280) & (TRASH - 1))
            idx_c[eb, j // 8, pl.ds((j % 8) * 16, 16)] = jnp.where(inb, loc, tr)
          return ()
        lax.fori_loop(0, CE // 64, idx_body, ())

        sdescs[eb] = [
            pltpu.async_copy(ones_v, acc_sh.at[idx_c.at[eb, j]], sem, add=True)
            for j in range(CROWS)
        ]
      for descs in sdescs.values():
        for d in descs:
          d.wait()
      plsc.subcore_barrier()

      pltpu.sync_copy(
          acc_sh.at[pl.ds(s * OSLICE, OSLICE)],
          out_hbm.at[pl.ds(base_flat + s * OSLICE, OSLICE)],
      )
      if p == 0:
        zero_own_slices()
      plsc.subcore_barrier()

  return build(edge_index.reshape(2 * E))


def _tc_forward_body(A_ref, x_ref, Wc_ref, Wf_ref, P_ref, o_ref):
  f32 = jnp.float32
  A5 = A_ref[0]
  Ac = [A5[tc].reshape(NPP, H) for tc in range(5)]
  h = x_ref[0]

  def matA(u):
    acc = jnp.dot(Ac[0], u[0:H], preferred_element_type=f32)
    for tc in range(1, 5):
      acc = acc + jnp.dot(Ac[tc], u[tc * H:(tc + 1) * H],
                          preferred_element_type=f32)
    return acc

  rowi = lax.broadcasted_iota(jnp.int32, (NPP, 1), 0)
  colj = lax.broadcasted_iota(jnp.int32, (1, NPP), 1)
  ident = (lax.broadcasted_iota(jnp.int32, (NPP, NPP), 0) ==
           lax.broadcasted_iota(jnp.int32, (NPP, NPP), 1)).astype(f32)
  m = (rowi < NPG).astype(f32)
  bn_scale = 1.0 / math.sqrt(1.0 + EPS)

  for st in range(4):
    k = K_LIST[st]
    Wc = Wc_ref[st]
    Wf = Wf_ref[st]
    bc = P_ref[pl.ds(st, 1), :]
    bf = P_ref[pl.ds(4 + st, 1), :]
    gg = P_ref[pl.ds(8 + st, 1), :]
    be = P_ref[pl.ds(12 + st, 1), :]
    ws = P_ref[pl.ds(16 + st, 1), :]
    bs = P_ref[20 + st, 0]

    degv = matA(m)
    deg = m * (degv + 1.0)
    dinv = jnp.where(deg > 0, 1.0 / jnp.sqrt(jnp.where(deg > 0, deg, 1.0)),
                     0.0)

    hw = jnp.dot(h, Wc, preferred_element_type=f32)
    v = matA(dinv * hw)
    conv = dinv * v + (dinv * dinv) * hw + bc

    h2 = jnp.maximum(jnp.dot(conv, Wf, preferred_element_type=f32) + bf, 0.0)
    h2 = h2 * (bn_scale * gg) + be

    hs = jnp.sum(h2 * ws, axis=1, keepdims=True)
    vs = matA(dinv * hs)
    sc = jnp.tanh(dinv * vs + (dinv * dinv) * hs + bs)

    sm = jnp.where(m > 0, sc, -2.0)
    smT = lax.dot_general(sm, ident, (((0,), (0,)), ((), ())))
    cmp = (smT > sm) | ((smT == sm) & (colj < rowi))
    rank = jnp.sum(cmp.astype(f32), axis=1, keepdims=True)
    m = (rank < k).astype(f32)
    h = h2 * sc * m

  sums = jnp.sum(h, axis=0, keepdims=True)
  cnt = jnp.sum(m)
  o_ref[0] = sums / cnt


def _tc_forward(A3, xp, Wcs, Wfs, P):
  return pl.pallas_call(
      _tc_forward_body,
      grid=(G,),
      in_specs=[
          pl.BlockSpec((1, 5, NPP // 8, 8, H), lambda g: (g, 0, 0, 0, 0)),
          pl.BlockSpec((1, NPP, H), lambda g: (g, 0, 0)),
          pl.BlockSpec((4, H, H), lambda g: (0, 0, 0)),
          pl.BlockSpec((4, H, H), lambda g: (0, 0, 0)),
          pl.BlockSpec((24, H), lambda g: (0, 0)),
      ],
      out_specs=pl.BlockSpec((1, 1, H), lambda g: (g, 0, 0)),
      out_shape=jax.ShapeDtypeStruct((G, 1, H), jnp.float32),
  )(A3, xp, Wcs, Wfs, P)


def kernel(x, edge_index, batch,
           Wc1, bc1, Wf1, bf1, g1, be1, Ws1, bs1,
           Wc2, bc2, Wf2, bf2, g2, be2, Ws2, bs2,
           Wc3, bc3, Wf3, bf3, g3, be3, Ws3, bs3,
           Wc4, bc4, Wf4, bf4, g4, be4, Ws4, bs4):
  del batch
  A_flat = _sc_build_adjacency(edge_index)
  A3 = A_flat.reshape(G, 5, NPP // 8, 8, H)

  xp = jnp.pad(x.reshape(G, NPG, H), ((0, 0), (0, NPP - NPG), (0, 0)))

  Wcs = jnp.stack([Wc1, Wc2, Wc3, Wc4])
  Wfs = jnp.stack([Wf1, Wf2, Wf3, Wf4])
  P = jnp.stack([
      bc1, bc2, bc3, bc4,
      bf1, bf2, bf3, bf4,
      g1, g2, g3, g4,
      be1, be2, be3, be4,
      Ws1[:, 0], Ws2[:, 0], Ws3[:, 0], Ws4[:, 0],
      jnp.full((H,), bs1[0]), jnp.full((H,), bs2[0]),
      jnp.full((H,), bs3[0]), jnp.full((H,), bs4[0]),
  ])

  out = _tc_forward(A3, xp, Wcs, Wfs, P)
  return out.reshape(G, H)


# speedup vs baseline: 1.1660x; 1.1660x over previous
import functools
import math

import jax
import jax.numpy as jnp
from jax import lax
from jax.experimental import pallas as pl
from jax.experimental.pallas import tpu as pltpu
from jax.experimental.pallas import tpu_sc as plsc

N = 10000
E = 320000
G = 16
NPG = 625
NPP = 640
H = 128
EPS = 1e-5
K_LIST = [313, 157, 79, 40]

GSZ = NPP * NPP
CHUNK = 4 * GSZ
TRASH = 8192
REGION = CHUNK + TRASH
OSLICE = CHUNK // 16
ZBUF = 3200
TSLICE = TRASH // 16

EPT = E // 16
CE = 4096
NCH = 5
CE_TAIL = EPT - (NCH - 1) * CE
CROWS = CE // 128
MAGIC = 26844


def _sc_build_adjacency(edge_flat, half):
  mesh = plsc.VectorSubcoreMesh(core_axis_name="c", subcore_axis_name="s")

  @functools.partial(
      pl.kernel,
      out_type=jax.ShapeDtypeStruct((G * GSZ // 2,), jnp.float32),
      mesh=mesh,
      scratch_types=[
          pltpu.VMEM((2 * CE,), jnp.int32),
          pltpu.VMEM((2 * CE,), jnp.int32),
          pltpu.VMEM((2, CROWS, 128), jnp.int32),
          pltpu.VMEM((128,), jnp.float32),
          pltpu.VMEM((ZBUF,), jnp.float32),
          pltpu.VMEM_SHARED((REGION,), jnp.float32),
          pltpu.SemaphoreType.DMA,
          pltpu.SemaphoreType.DMA,
      ],
  )
  def build(edge_hbm, out_hbm, src_c, dst_c, idx_c,
            ones_v, zeros_v, acc_sh, sem, esem):
    c = lax.axis_index("c")
    s = lax.axis_index("s")
    lane = lax.iota(jnp.int32, 16)

    for i in range(8):
      ones_v[pl.ds(i * 16, 16)] = jnp.ones((16,), jnp.float32)

    def zfill(i, _):
      zeros_v[pl.ds(i * 16, 16)] = jnp.zeros((16,), jnp.float32)
      return ()
    lax.fori_loop(0, ZBUF // 16, zfill, ())

    def zero_own_slices():
      for z in range(32):
        pltpu.sync_copy(zeros_v, acc_sh.at[pl.ds(s * OSLICE + z * ZBUF, ZBUF)])
      pltpu.sync_copy(zeros_v.at[pl.ds(0, TSLICE)],
                      acc_sh.at[pl.ds(CHUNK + s * TSLICE, TSLICE)])

    def fetch(ci, eb):
      csz = CE_TAIL if ci == NCH - 1 else CE
      ebase = s * EPT + ci * CE
      return [
          pltpu.async_copy(edge_hbm.at[pl.ds(ebase, csz)],
                           src_c.at[pl.ds(eb * CE, csz)], esem),
          pltpu.async_copy(edge_hbm.at[pl.ds(E + ebase, csz)],
                           dst_c.at[pl.ds(eb * CE, csz)], esem),
      ]

    zero_own_slices()
    plsc.subcore_barrier()

    for p in range(1):
      chunk = half * 2 + c
      base_flat = chunk * CHUNK

      efetch = {0: fetch(0, 0)}
      sdescs = {}
      for ci in range(NCH):
        eb = ci % 2
        for d in efetch.pop(ci % 2):
          d.wait()
        if ci + 1 < NCH:
          efetch[(ci + 1) % 2] = fetch(ci + 1, (ci + 1) % 2)

        for d in sdescs.pop(eb, []):
          d.wait()

        def idx_body(jj, _):
          for u in range(4):
            j = jj * 4 + u
            e = j * 16
            sv = src_c[pl.ds(eb * CE + e, 16)]
            dv = dst_c[pl.ds(eb * CE + e, 16)]
            g = (dv * MAGIC) >> 24
            ld = dv - g * NPG
            ls = sv - ((sv * MAGIC) >> 24) * NPG
            tiled = ((ls >> 7) * 81920 + ((ld >> 3) << 10)
                     + ((ld & 7) << 7) + (ls & 127))
            flat = g * GSZ + tiled
            loc = flat - base_flat
            eid = ci * CE + e + lane
            inb = (eid < EPT) & (loc >= 0) & (loc < CHUNK)
            tr = CHUNK + ((eid + s * 1280) & (TRASH - 1))
            idx_c[eb, j // 8, pl.ds((j % 8) * 16, 16)] = jnp.where(inb, loc, tr)
          return ()
        lax.fori_loop(0, CE // 64, idx_body, ())

        sdescs[eb] = [
            pltpu.async_copy(ones_v, acc_sh.at[idx_c.at[eb, j]], sem, add=True)
            for j in range(CROWS)
        ]
      for descs in sdescs.values():
        for d in descs:
          d.wait()
      plsc.subcore_barrier()

      pltpu.sync_copy(
          acc_sh.at[pl.ds(s * OSLICE, OSLICE)],
          out_hbm.at[pl.ds(c * CHUNK + s * OSLICE, OSLICE)],
      )

  return build(edge_flat)


def _tc_forward_body(A_ref, x_ref, Wc_ref, Wf_ref, P_ref, o_ref):
  f32 = jnp.float32
  A5 = A_ref[0]
  Ac = [A5[tc].reshape(NPP, H) for tc in range(5)]
  h = x_ref[0]

  def matA(u):
    acc = jnp.dot(Ac[0], u[0:H], preferred_element_type=f32)
    for tc in range(1, 5):
      acc = acc + jnp.dot(Ac[tc], u[tc * H:(tc + 1) * H],
                          preferred_element_type=f32)
    return acc

  rowi = lax.broadcasted_iota(jnp.int32, (NPP, 1), 0)
  colj = lax.broadcasted_iota(jnp.int32, (1, NPP), 1)
  tri = colj < rowi
  ident = (lax.broadcasted_iota(jnp.int32, (NPP, NPP), 0) ==
           lax.broadcasted_iota(jnp.int32, (NPP, NPP), 1)).astype(f32)
  m = (rowi < NPG).astype(f32)
  bn_scale = 1.0 / math.sqrt(1.0 + EPS)

  for st in range(4):
    k = K_LIST[st]
    Wc = Wc_ref[st]
    Wf = Wf_ref[st]
    bc = P_ref[pl.ds(st, 1), :]
    bf = P_ref[pl.ds(4 + st, 1), :]
    gg = P_ref[pl.ds(8 + st, 1), :]
    be = P_ref[pl.ds(12 + st, 1), :]
    ws = P_ref[pl.ds(16 + st, 1), :]
    bs = P_ref[20 + st, 0]

    degv = matA(m)
    deg = m * (degv + 1.0)
    dinv = m * lax.rsqrt(deg + (1.0 - m))

    hw = jnp.dot(h, Wc, preferred_element_type=f32)
    v = matA(dinv * hw)
    conv = dinv * v + (dinv * dinv) * hw + bc

    h2 = jnp.maximum(jnp.dot(conv, Wf, preferred_element_type=f32) + bf, 0.0)
    h2 = h2 * (bn_scale * gg) + be

    hs = jnp.sum(h2 * ws, axis=1, keepdims=True)
    vs = matA(dinv * hs)
    sc = jnp.tanh(dinv * vs + (dinv * dinv) * hs + bs)

    sm = jnp.where(m > 0, sc, -2.0)
    smT = lax.dot_general(sm, ident, (((0,), (0,)), ((), ())))
    cmp = (smT > sm) | ((smT == sm) & tri)
    rank = jnp.sum(cmp.astype(f32), axis=1, keepdims=True)
    m = (rank < k).astype(f32)
    h = h2 * sc * m

  sums = jnp.sum(h, axis=0, keepdims=True)
  cnt = jnp.sum(m)
  o_ref[0] = sums / cnt


def _tc_forward(A3, xp, Wcs, Wfs, P):
  ng = A3.shape[0]
  return pl.pallas_call(
      _tc_forward_body,
      grid=(ng,),
      in_specs=[
          pl.BlockSpec((1, 5, NPP // 8, 8, H), lambda g: (g, 0, 0, 0, 0)),
          pl.BlockSpec((1, NPP, H), lambda g: (g, 0, 0)),
          pl.BlockSpec((4, H, H), lambda g: (0, 0, 0)),
          pl.BlockSpec((4, H, H), lambda g: (0, 0, 0)),
          pl.BlockSpec((24, H), lambda g: (0, 0)),
      ],
      out_specs=pl.BlockSpec((1, 1, H), lambda g: (g, 0, 0)),
      out_shape=jax.ShapeDtypeStruct((ng, 1, H), jnp.float32),
  )(A3, xp, Wcs, Wfs, P)


def kernel(x, edge_index, batch,
           Wc1, bc1, Wf1, bf1, g1, be1, Ws1, bs1,
           Wc2, bc2, Wf2, bf2, g2, be2, Ws2, bs2,
           Wc3, bc3, Wf3, bf3, g3, be3, Ws3, bs3,
           Wc4, bc4, Wf4, bf4, g4, be4, Ws4, bs4):
  del batch
  edge_flat = edge_index.reshape(2 * E)
  xp = jnp.pad(x.reshape(G, NPG, H), ((0, 0), (0, NPP - NPG), (0, 0)))

  Wcs = jnp.stack([Wc1, Wc2, Wc3, Wc4])
  Wfs = jnp.stack([Wf1, Wf2, Wf3, Wf4])
  P = jnp.stack([
      bc1, bc2, bc3, bc4,
      bf1, bf2, bf3, bf4,
      g1, g2, g3, g4,
      be1, be2, be3, be4,
      Ws1[:, 0], Ws2[:, 0], Ws3[:, 0], Ws4[:, 0],
      jnp.full((H,), bs1[0]), jnp.full((H,), bs2[0]),
      jnp.full((H,), bs3[0]), jnp.full((H,), bs4[0]),
  ])

  outs = []
  for half in range(2):
    A_flat = _sc_build_adjacency(edge_flat, half)
    A3 = A_flat.reshape(G // 2, 5, NPP // 8, 8, H)
    outs.append(_tc_forward(A3, xp[half * 8:half * 8 + 8], Wcs, Wfs, P))
  return jnp.concatenate(outs, axis=0).reshape(G, H)
